# Initial kernel scaffold; baseline (speedup 1.0000x reference)
import functools

import jax
import jax.numpy as jnp
from jax import lax
from jax.experimental import pallas as pl
from jax.experimental.pallas import tpu as pltpu
from jax.experimental.pallas import tpu_sc as plsc

N = 10000
NC = 2
NS = 16
NW = NC * NS
CH = 128
K = 80
EP = NW * K * CH
RB = 626
NPAD = NS * RB
RBLK = 2000


def _edge_kernel(Dm, with_count):
  mesh = plsc.VectorSubcoreMesh(
      core_axis_name="c", subcore_axis_name="s", num_cores=NC, num_subcores=NS)
  out_type = [jax.ShapeDtypeStruct((NC, NPAD, Dm), jnp.float32)]
  scratch = [
      pltpu.VMEM((K, CH), jnp.int32),
      pltpu.VMEM((K, CH), jnp.int32),
      pltpu.VMEM((CH, Dm), jnp.float32),
      pltpu.VMEM((CH, Dm), jnp.float32),
      pltpu.VMEM_SHARED((NPAD, Dm), jnp.float32),
      pltpu.SemaphoreType.DMA,
      pltpu.SemaphoreType.DMA,
  ]
  if with_count:
    out_type.append(jax.ShapeDtypeStruct((NC, NPAD, 8), jnp.float32))
    scratch += [
        pltpu.VMEM((CH, 8), jnp.float32),
        pltpu.VMEM_SHARED((NPAD, 8), jnp.float32),
    ]

  def body(*refs):
    if with_count:
      (tab, srcp, dstp, z2, z8, ones_h,
       out_agg, out_cnt,
       src_v, dst_v, rows0, rows1, agg_sh, sem0, sem1, ones_v, cnt_sh) = refs
    else:
      (tab, srcp, dstp, z2,
       out_agg,
       src_v, dst_v, rows0, rows1, agg_sh, sem0, sem1) = refs
    c = lax.axis_index("c")
    s = lax.axis_index("s")
    wid = c * NS + s
    r0 = s * RB
    pltpu.sync_copy(z2.at[pl.ds(r0, RB), :], agg_sh.at[pl.ds(r0, RB), :])
    if with_count:
      pltpu.sync_copy(z8.at[pl.ds(r0, RB), :], cnt_sh.at[pl.ds(r0, RB), :])
      pltpu.sync_copy(ones_h, ones_v)
    plsc.subcore_barrier()

    base = wid * K
    pltpu.sync_copy(srcp.at[pl.ds(base, K), :], src_v)
    pltpu.sync_copy(dstp.at[pl.ds(base, K), :], dst_v)

    pltpu.async_copy(tab.at[src_v.at[0]], rows0, sem0)

    def pair(t, carry):
      jj = 2 * t
      pltpu.make_async_copy(tab.at[src_v.at[jj]], rows0, sem0).wait()
      pltpu.async_copy(tab.at[src_v.at[jj + 1]], rows1, sem1)
      pltpu.sync_copy(rows0, agg_sh.at[dst_v.at[jj]], add=True)
      if with_count:
        pltpu.sync_copy(ones_v, cnt_sh.at[dst_v.at[jj]], add=True)
      pltpu.make_async_copy(tab.at[src_v.at[jj + 1]], rows1, sem1).wait()

      @pl.when(jj + 2 < K)
      def _():
        pltpu.async_copy(tab.at[src_v.at[jj + 2]], rows0, sem0)

      pltpu.sync_copy(rows1, agg_sh.at[dst_v.at[jj + 1]], add=True)
      if with_count:
        pltpu.sync_copy(ones_v, cnt_sh.at[dst_v.at[jj + 1]], add=True)
      return carry

    lax.fori_loop(0, K // 2, pair, 0)
    plsc.subcore_barrier()
    pltpu.sync_copy(agg_sh.at[pl.ds(r0, RB), :],
                    out_agg.at[c].at[pl.ds(r0, RB), :])
    if with_count:
      pltpu.sync_copy(cnt_sh.at[pl.ds(r0, RB), :],
                      out_cnt.at[c].at[pl.ds(r0, RB), :])

  return pl.kernel(body, out_type=out_type, mesh=mesh, scratch_types=scratch)


def _tc1_body(x, W1, b1, out):
  out[...] = jnp.maximum(
      jnp.dot(x[...], W1[...], preferred_element_type=jnp.float32) + b1[...],
      0.0)


def _tc2_body(agg0, agg1, cnt0, cnt1, x, W2, Wr, b2, W1n, b1n,
              h1, p2, inv):
  iv = 1.0 / jnp.maximum(cnt0[...] + cnt1[...], 1.0)
  mean = (agg0[...] + agg1[...]) * iv
  h = jnp.maximum(
      jnp.dot(mean, W2[...], preferred_element_type=jnp.float32)
      + jnp.dot(x[...], Wr[...], preferred_element_type=jnp.float32)
      + b2[...], 0.0)
  h1[...] = h
  p2[...] = jnp.maximum(
      jnp.dot(h, W1n[...], preferred_element_type=jnp.float32) + b1n[...], 0.0)
  inv[...] = iv


def _tc3_body(agg0, agg1, inv, h1, W2, Wr, b2, lW, lb, y):
  mean = (agg0[...] + agg1[...]) * inv[...]
  h = jnp.maximum(
      jnp.dot(mean, W2[...], preferred_element_type=jnp.float32)
      + jnp.dot(h1[...], Wr[...], preferred_element_type=jnp.float32)
      + b2[...], 0.0)
  y[...] = jnp.dot(h, lW[...], preferred_element_type=jnp.float32) + lb[...]


def _row_spec(d):
  return pl.BlockSpec((RBLK, d), lambda i: (i, 0))


def _full_spec(a, b):
  return pl.BlockSpec((a, b), lambda i: (0, 0))


@jax.jit
def kernel(x, edge_index, c1_W1, c1_b1, c1_W2, c1_b2, c1_Wr,
           c2_W1, c2_b1, c2_W2, c2_b2, c2_Wr, lin_W, lin_b):
  E = edge_index.shape[1]
  pad = EP - E
  src = jnp.concatenate([edge_index[0], jnp.zeros((pad,), jnp.int32)])
  dst = jnp.concatenate([edge_index[1], jnp.full((pad,), N, jnp.int32)])
  srcp = src.reshape(NW * K, CH)
  dstp = dst.reshape(NW * K, CH)
  z64 = jnp.zeros((NPAD, 64), jnp.float32)
  z32 = jnp.zeros((NPAD, 32), jnp.float32)
  z8 = jnp.zeros((NPAD, 8), jnp.float32)
  ones8 = jnp.ones((CH, 8), jnp.float32)

  grid = N // RBLK

  p1 = pl.pallas_call(
      _tc1_body,
      grid=(grid,),
      in_specs=[_row_spec(128), _full_spec(128, 64), _full_spec(1, 64)],
      out_specs=_row_spec(64),
      out_shape=jax.ShapeDtypeStruct((N, 64), jnp.float32),
  )(x, c1_W1, c1_b1.reshape(1, 64))

  agg1p, cntp = _edge_kernel(64, True)(p1, srcp, dstp, z64, z8, ones8)
  agg1_0 = agg1p[0, :N, :]
  agg1_1 = agg1p[1, :N, :]
  cnt0 = cntp[0, :N, 0:1]
  cnt1 = cntp[1, :N, 0:1]

  h1, p2, inv = pl.pallas_call(
      _tc2_body,
      grid=(grid,),
      in_specs=[_row_spec(64), _row_spec(64), _row_spec(1), _row_spec(1),
                _row_spec(128), _full_spec(64, 64), _full_spec(128, 64),
                _full_spec(1, 64), _full_spec(64, 32), _full_spec(1, 32)],
      out_specs=[_row_spec(64), _row_spec(32), _row_spec(1)],
      out_shape=[jax.ShapeDtypeStruct((N, 64), jnp.float32),
                 jax.ShapeDtypeStruct((N, 32), jnp.float32),
                 jax.ShapeDtypeStruct((N, 1), jnp.float32)],
  )(agg1_0, agg1_1, cnt0, cnt1, x, c1_W2, c1_Wr, c1_b2.reshape(1, 64),
    c2_W1, c2_b1.reshape(1, 32))

  (agg2p,) = _edge_kernel(32, False)(p2, srcp, dstp, z32)
  agg2_0 = agg2p[0, :N, :]
  agg2_1 = agg2p[1, :N, :]

  y = pl.pallas_call(
      _tc3_body,
      grid=(grid,),
      in_specs=[_row_spec(32), _row_spec(32), _row_spec(1), _row_spec(64),
                _full_spec(32, 32), _full_spec(64, 32), _full_spec(1, 32),
                _full_spec(32, 1), _full_spec(1, 1)],
      out_specs=_row_spec(1),
      out_shape=jax.ShapeDtypeStruct((N, 1), jnp.float32),
  )(agg2_0, agg2_1, inv, h1, c2_W2, c2_Wr, c2_b2.reshape(1, 32),
    lin_W, lin_b.reshape(1, 1))

  return y


# speedup vs baseline: 5.8639x; 5.8639x over previous
import functools

import jax
import jax.numpy as jnp
from jax import lax
from jax.experimental import pallas as pl
from jax.experimental.pallas import tpu as pltpu
from jax.experimental.pallas import tpu_sc as plsc

N = 10000
NC = 2
NS = 16
NW = NC * NS
CH = 128
K = 80
EP = NW * K * CH
RB = 632
NPAD = NS * RB
RBLK = 2000


def _edge_kernel(Dm, with_count):
  mesh = plsc.VectorSubcoreMesh(
      core_axis_name="c", subcore_axis_name="s", num_cores=NC, num_subcores=NS)
  out_type = [jax.ShapeDtypeStruct((NC, NPAD, Dm), jnp.float32)]
  scratch = [
      pltpu.VMEM((K, CH), jnp.int32),
      pltpu.VMEM((K, CH), jnp.int32),
      pltpu.VMEM((CH, Dm), jnp.float32),
      pltpu.VMEM((CH, Dm), jnp.float32),
      pltpu.VMEM_SHARED((NPAD, Dm), jnp.float32),
      pltpu.SemaphoreType.DMA,
      pltpu.SemaphoreType.DMA,
  ]
  if with_count:
    out_type.append(jax.ShapeDtypeStruct((NC, NPAD, 8), jnp.float32))
    scratch += [
        pltpu.VMEM((CH, 8), jnp.float32),
        pltpu.VMEM_SHARED((NPAD, 8), jnp.float32),
    ]

  def body(*refs):
    if with_count:
      (tab, srcp, dstp, z2, z8, ones_h,
       out_agg, out_cnt,
       src_v, dst_v, rows0, rows1, agg_sh, sem0, sem1, ones_v, cnt_sh) = refs
    else:
      (tab, srcp, dstp, z2,
       out_agg,
       src_v, dst_v, rows0, rows1, agg_sh, sem0, sem1) = refs
    c = lax.axis_index("c")
    s = lax.axis_index("s")
    wid = c * NS + s
    r0 = s * RB
    pltpu.sync_copy(z2.at[pl.ds(r0, RB), :], agg_sh.at[pl.ds(r0, RB), :])
    if with_count:
      pltpu.sync_copy(z8.at[pl.ds(r0, RB), :], cnt_sh.at[pl.ds(r0, RB), :])
      pltpu.sync_copy(ones_h, ones_v)
    plsc.subcore_barrier()

    base = wid * K
    pltpu.sync_copy(srcp.at[pl.ds(base, K), :], src_v)
    pltpu.sync_copy(dstp.at[pl.ds(base, K), :], dst_v)

    pltpu.async_copy(tab.at[src_v.at[0]], rows0, sem0)

    def pair(t, carry):
      jj = 2 * t
      pltpu.make_async_copy(tab.at[src_v.at[jj]], rows0, sem0).wait()
      pltpu.async_copy(tab.at[src_v.at[jj + 1]], rows1, sem1)
      pltpu.sync_copy(rows0, agg_sh.at[dst_v.at[jj]], add=True)
      if with_count:
        pltpu.sync_copy(ones_v, cnt_sh.at[dst_v.at[jj]], add=True)
      pltpu.make_async_copy(tab.at[src_v.at[jj + 1]], rows1, sem1).wait()

      @pl.when(jj + 2 < K)
      def _():
        pltpu.async_copy(tab.at[src_v.at[jj + 2]], rows0, sem0)

      pltpu.sync_copy(rows1, agg_sh.at[dst_v.at[jj + 1]], add=True)
      if with_count:
        pltpu.sync_copy(ones_v, cnt_sh.at[dst_v.at[jj + 1]], add=True)
      return carry

    lax.fori_loop(0, K // 2, pair, 0)
    plsc.subcore_barrier()
    pltpu.sync_copy(agg_sh.at[pl.ds(r0, RB), :],
                    out_agg.at[c].at[pl.ds(r0, RB), :])
    if with_count:
      pltpu.sync_copy(cnt_sh.at[pl.ds(r0, RB), :],
                      out_cnt.at[c].at[pl.ds(r0, RB), :])

  return pl.kernel(body, out_type=out_type, mesh=mesh, scratch_types=scratch,
                   compiler_params=pltpu.CompilerParams(
                       use_tc_tiling_on_sc=False))


def _tc1_body(x, W1, b1, out):
  out[...] = jnp.maximum(
      jnp.dot(x[...], W1[...], preferred_element_type=jnp.float32) + b1[...],
      0.0)


def _tc2_body(agg0, agg1, cnt0, cnt1, x, W2, Wr, b2, W1n, b1n,
              h1, p2, inv):
  iv = 1.0 / jnp.maximum(cnt0[...] + cnt1[...], 1.0)
  mean = (agg0[...] + agg1[...]) * iv
  h = jnp.maximum(
      jnp.dot(mean, W2[...], preferred_element_type=jnp.float32)
      + jnp.dot(x[...], Wr[...], preferred_element_type=jnp.float32)
      + b2[...], 0.0)
  h1[...] = h
  p2[...] = jnp.maximum(
      jnp.dot(h, W1n[...], preferred_element_type=jnp.float32) + b1n[...], 0.0)
  inv[...] = iv


def _tc3_body(agg0, agg1, inv, h1, W2, Wr, b2, lW, lb, y):
  mean = (agg0[...] + agg1[...]) * inv[...]
  h = jnp.maximum(
      jnp.dot(mean, W2[...], preferred_element_type=jnp.float32)
      + jnp.dot(h1[...], Wr[...], preferred_element_type=jnp.float32)
      + b2[...], 0.0)
  y[...] = jnp.dot(h, lW[...], preferred_element_type=jnp.float32) + lb[...]


def _row_spec(d):
  return pl.BlockSpec((RBLK, d), lambda i: (i, 0))


def _full_spec(a, b):
  return pl.BlockSpec((a, b), lambda i: (0, 0))


@jax.jit
def kernel(x, edge_index, c1_W1, c1_b1, c1_W2, c1_b2, c1_Wr,
           c2_W1, c2_b1, c2_W2, c2_b2, c2_Wr, lin_W, lin_b):
  E = edge_index.shape[1]
  pad = EP - E
  src = jnp.concatenate([edge_index[0], jnp.zeros((pad,), jnp.int32)])
  dst = jnp.concatenate([edge_index[1], jnp.full((pad,), N, jnp.int32)])
  srcp = src.reshape(NW * K, CH)
  dstp = dst.reshape(NW * K, CH)
  z64 = jnp.zeros((NPAD, 64), jnp.float32)
  z32 = jnp.zeros((NPAD, 32), jnp.float32)
  z8 = jnp.zeros((NPAD, 8), jnp.float32)
  ones8 = jnp.ones((CH, 8), jnp.float32)

  grid = N // RBLK

  p1 = pl.pallas_call(
      _tc1_body,
      grid=(grid,),
      in_specs=[_row_spec(128), _full_spec(128, 64), _full_spec(1, 64)],
      out_specs=_row_spec(64),
      out_shape=jax.ShapeDtypeStruct((N, 64), jnp.float32),
  )(x, c1_W1, c1_b1.reshape(1, 64))

  agg1p, cntp = _edge_kernel(64, True)(p1, srcp, dstp, z64, z8, ones8)
  agg1_0 = agg1p[0, :N, :]
  agg1_1 = agg1p[1, :N, :]
  cnt0 = cntp[0, :N, 0:1]
  cnt1 = cntp[1, :N, 0:1]

  h1, p2, inv = pl.pallas_call(
      _tc2_body,
      grid=(grid,),
      in_specs=[_row_spec(64), _row_spec(64), _row_spec(1), _row_spec(1),
                _row_spec(128), _full_spec(64, 64), _full_spec(128, 64),
                _full_spec(1, 64), _full_spec(64, 32), _full_spec(1, 32)],
      out_specs=[_row_spec(64), _row_spec(32), _row_spec(1)],
      out_shape=[jax.ShapeDtypeStruct((N, 64), jnp.float32),
                 jax.ShapeDtypeStruct((N, 32), jnp.float32),
                 jax.ShapeDtypeStruct((N, 1), jnp.float32)],
  )(agg1_0, agg1_1, cnt0, cnt1, x, c1_W2, c1_Wr, c1_b2.reshape(1, 64),
    c2_W1, c2_b1.reshape(1, 32))

  (agg2p,) = _edge_kernel(32, False)(p2, srcp, dstp, z32)
  agg2_0 = agg2p[0, :N, :]
  agg2_1 = agg2p[1, :N, :]

  y = pl.pallas_call(
      _tc3_body,
      grid=(grid,),
      in_specs=[_row_spec(32), _row_spec(32), _row_spec(1), _row_spec(64),
                _full_spec(32, 32), _full_spec(64, 32), _full_spec(1, 32),
                _full_spec(32, 1), _full_spec(1, 1)],
      out_specs=_row_spec(1),
      out_shape=jax.ShapeDtypeStruct((N, 1), jnp.float32),
  )(agg2_0, agg2_1, inv, h1, c2_W2, c2_Wr, c2_b2.reshape(1, 32),
    lin_W, lin_b.reshape(1, 1))

  return y
